# baseline (device time: 462805 ns/iter reference)
import functools

import jax
import jax.numpy as jnp
from jax import lax
from jax.experimental import pallas as pl
from jax.experimental.pallas import tpu as pltpu

N_DEV = 16


def kernel(x, w_mat, scale_x, scale_w):
    m_total, k_blk = x.shape
    _, n = w_mat.shape
    m_blk = m_total // N_DEV

    x = x.astype(jnp.float8_e4m3fn)
    w_mat = w_mat.astype(jnp.float8_e5m2)

    def body(x_ref, w_ref, sx_ref, sw_ref, out_ref,
             wbuf, xtiles, wsend, wrecv, xsend, xrecv):
        me = lax.axis_index("i")
        right = lax.rem(me + 1, N_DEV)

        def mod(v):
            return lax.rem(v + N_DEV, N_DEV)

        barrier = pltpu.get_barrier_semaphore()
        for k in range(1, N_DEV):
            pl.semaphore_signal(barrier, inc=1, device_id=(mod(me + k),),
                                device_id_type=pl.DeviceIdType.MESH)
        pl.semaphore_wait(barrier, N_DEV - 1)

        x_sends = []
        for k in range(1, N_DEV):
            t = mod(me + k)
            rdma = pltpu.make_async_remote_copy(
                src_ref=x_ref.at[pl.ds(t * m_blk, m_blk), :],
                dst_ref=xtiles.at[me],
                send_sem=xsend.at[t],
                recv_sem=xrecv.at[me],
                device_id=(t,),
                device_id_type=pl.DeviceIdType.MESH,
            )
            rdma.start()
            x_sends.append(rdma)

        n_half = n // 2

        def wait_x_recv(origin):
            pltpu.make_async_remote_copy(
                src_ref=x_ref.at[pl.ds(0, m_blk), :], dst_ref=xtiles.at[origin],
                send_sem=xsend.at[origin], recv_sem=xrecv.at[origin],
                device_id=(right,), device_id_type=pl.DeviceIdType.MESH,
            ).wait_recv()

        def w_half_desc(src, origin, j):
            return pltpu.make_async_remote_copy(
                src_ref=src.at[:, pl.ds(j * n_half, n_half)],
                dst_ref=wbuf.at[origin, :, pl.ds(j * n_half, n_half)],
                send_sem=wsend.at[origin, j],
                recv_sem=wrecv.at[origin, j],
                device_id=(right,),
                device_id_type=pl.DeviceIdType.MESH,
            )

        w_descs = []
        for j in range(2):
            d = w_half_desc(w_ref, me, j)
            d.start()
            w_descs.append(d)

        x_own = x_ref[pl.ds(me * m_blk, m_blk), :].astype(jnp.bfloat16)
        out_ref[:, :] = jnp.dot(x_own, w_ref[:, :].astype(jnp.bfloat16),
                                preferred_element_type=jnp.float32)

        for h in range(N_DEV - 1):
            o = mod(me - h - 1)
            wait_x_recv(o)
            for j in range(2):
                w_half_desc(wbuf.at[o], o, j).wait_recv()
                if h < N_DEV - 2:
                    d = w_half_desc(wbuf.at[o], o, j)
                    d.start()
                    w_descs.append(d)
                xt = xtiles[o].astype(jnp.bfloat16)
                wh = wbuf[o, :, pl.ds(j * n_half, n_half)].astype(jnp.bfloat16)
                out_ref[:, pl.ds(j * n_half, n_half)] += jnp.dot(
                    xt, wh, preferred_element_type=jnp.float32)

        scale = sx_ref[0] * sw_ref[0]
        y = out_ref[:, :] * scale
        out_ref[:, :] = y / (1.0 + jnp.exp(-jnp.clip(y, -60.0, 60.0)))

        for rdma in x_sends + w_descs:
            rdma.wait_send()

        @functools.partial(pl.run_scoped, sem2=pltpu.SemaphoreType.REGULAR)
        def _(sem2):
            for k in range(1, N_DEV):
                pl.semaphore_signal(sem2, inc=1, device_id=(mod(me + k),),
                                    device_id_type=pl.DeviceIdType.MESH)
            pl.semaphore_wait(sem2, N_DEV - 1)

    return pl.pallas_call(
        body,
        out_shape=jax.ShapeDtypeStruct((m_blk, n), jnp.float32),
        in_specs=[
            pl.BlockSpec(memory_space=pltpu.VMEM),
            pl.BlockSpec(memory_space=pltpu.VMEM),
            pl.BlockSpec(memory_space=pltpu.SMEM),
            pl.BlockSpec(memory_space=pltpu.SMEM),
        ],
        out_specs=pl.BlockSpec(memory_space=pltpu.VMEM),
        scratch_shapes=[
            pltpu.VMEM((N_DEV, k_blk, n), w_mat.dtype),
            pltpu.VMEM((N_DEV, m_blk, k_blk), x.dtype),
            pltpu.SemaphoreType.DMA((N_DEV, 2)),
            pltpu.SemaphoreType.DMA((N_DEV, 2)),
            pltpu.SemaphoreType.DMA((N_DEV,)),
            pltpu.SemaphoreType.DMA((N_DEV,)),
        ],
        compiler_params=pltpu.CompilerParams(
            collective_id=0,
            vmem_limit_bytes=60 * 1024 * 1024,
        ),
    )(x, w_mat, scale_x, scale_w)
